# Initial kernel scaffold; baseline (speedup 1.0000x reference)
import jax
import jax.numpy as jnp
from jax.experimental import pallas as pl


def kernel(x, w1, w2, Wg1, bg1, Wg2, bg2):
    raise NotImplementedError("write your pallas kernel here")


# speedup vs baseline: 5.0635x; 5.0635x over previous
import jax
import jax.numpy as jnp
from jax.experimental import pallas as pl

_K = 21
_INL = 6.0
_NEG = -3.0e38


def _emb_body(x_ref, w1_ref, w2_ref, emb_ref):
    h = jnp.tanh(x_ref[...] * w1_ref[...]) * w2_ref[...]
    nrm = jnp.sqrt(jnp.sum(h * h, axis=1, keepdims=True))
    emb_ref[...] = h / jnp.maximum(nrm, 1e-12)


def _simtopk_body(emb_blk_ref, emb_all_ref, st_ref, rowsum_ref, colsum_ref):
    i = pl.program_id(0)
    a = emb_blk_ref[...]
    b = emb_all_ref[...]
    s0 = jax.lax.dot_general(a, b, (((1,), (1,)), ((), ())),
                             preferred_element_type=jnp.float32)
    n = s0.shape[1]
    col = jax.lax.broadcasted_iota(jnp.int32, s0.shape, 1)
    s = s0
    for _ in range(_K):
        m = jnp.max(s, axis=1, keepdims=True)
        am = jnp.min(jnp.where(s == m, col, n), axis=1, keepdims=True)
        s = jnp.where(col == am, _NEG, s)
    z = jnp.where(s == _NEG, s0, 0.0) * _INL - _INL
    st = jnp.where(z > 0, z + 1.0, jnp.exp(z))
    st_ref[...] = st
    rowsum_ref[...] = jnp.sum(st, axis=1, keepdims=True)
    onesv = jnp.full((st.shape[0], 1), 1.0, jnp.float32)
    csum = jax.lax.dot_general(st, onesv, (((0,), (0,)), ((), ())),
                               preferred_element_type=jnp.float32)

    @pl.when(i == 0)
    def _():
        colsum_ref[...] = jnp.zeros_like(colsum_ref)

    colsum_ref[...] += csum


def _prep_body(x_ref, wg1_ref, bg1_ref, rowsum_ref, colsum_ref, u_ref, mu1_ref):
    deg = 0.5 * (rowsum_ref[...] + colsum_ref[...])
    u = 1.0 / (jnp.sqrt(deg) + 1e-10)
    u_ref[...] = u
    m1 = jnp.dot(x_ref[...], wg1_ref[...],
                 preferred_element_type=jnp.float32) + bg1_ref[...]
    mu1_ref[...] = u * m1


def _spmm1_body(strow_ref, stcol_ref, mu1_ref, u_ref, wg2_ref, bg2_ref,
                mu2_ref):
    t = jax.lax.dot_general(strow_ref[...], mu1_ref[...],
                            (((1,), (0,)), ((), ())),
                            preferred_element_type=jnp.float32)
    t2 = jax.lax.dot_general(stcol_ref[...], mu1_ref[...],
                             (((0,), (0,)), ((), ())),
                             preferred_element_type=jnp.float32)
    h1 = jnp.maximum(u_ref[...] * (0.5 * (t + t2)), 0.0)
    m2 = jnp.dot(h1, wg2_ref[...],
                 preferred_element_type=jnp.float32) + bg2_ref[...]
    mu2_ref[...] = u_ref[...] * m2


def _spmm2_body(strow_ref, stcol_ref, mu2_ref, u_ref, out_ref):
    t = jax.lax.dot_general(strow_ref[...], mu2_ref[...],
                            (((1,), (0,)), ((), ())),
                            preferred_element_type=jnp.float32)
    t2 = jax.lax.dot_general(stcol_ref[...], mu2_ref[...],
                             (((0,), (0,)), ((), ())),
                             preferred_element_type=jnp.float32)
    out_ref[...] = u_ref[...] * (0.5 * (t + t2))


def kernel(x, w1, w2, Wg1, bg1, Wg2, bg2):
    n, d = x.shape
    hid = Wg1.shape[1]
    outd = Wg2.shape[1]
    br = 256 if n % 256 == 0 else n
    g = n // br

    emb = pl.pallas_call(
        _emb_body,
        out_shape=jax.ShapeDtypeStruct((n, d), jnp.float32),
    )(x, w1.reshape(1, d), w2.reshape(1, d))

    st, rowsum, colsum = pl.pallas_call(
        _simtopk_body,
        grid=(g,),
        in_specs=[pl.BlockSpec((br, d), lambda i: (i, 0)),
                  pl.BlockSpec((n, d), lambda i: (0, 0))],
        out_specs=[pl.BlockSpec((br, n), lambda i: (i, 0)),
                   pl.BlockSpec((br, 1), lambda i: (i, 0)),
                   pl.BlockSpec((n, 1), lambda i: (0, 0))],
        out_shape=[jax.ShapeDtypeStruct((n, n), jnp.float32),
                   jax.ShapeDtypeStruct((n, 1), jnp.float32),
                   jax.ShapeDtypeStruct((n, 1), jnp.float32)],
    )(emb, emb)

    u, mu1 = pl.pallas_call(
        _prep_body,
        out_shape=[jax.ShapeDtypeStruct((n, 1), jnp.float32),
                   jax.ShapeDtypeStruct((n, hid), jnp.float32)],
    )(x, Wg1, bg1.reshape(1, hid), rowsum, colsum)

    mu2 = pl.pallas_call(
        _spmm1_body,
        grid=(g,),
        in_specs=[pl.BlockSpec((br, n), lambda i: (i, 0)),
                  pl.BlockSpec((n, br), lambda i: (0, i)),
                  pl.BlockSpec((n, hid), lambda i: (0, 0)),
                  pl.BlockSpec((br, 1), lambda i: (i, 0)),
                  pl.BlockSpec((hid, outd), lambda i: (0, 0)),
                  pl.BlockSpec((1, outd), lambda i: (0, 0))],
        out_specs=pl.BlockSpec((br, outd), lambda i: (i, 0)),
        out_shape=jax.ShapeDtypeStruct((n, outd), jnp.float32),
    )(st, st, mu1, u, Wg2, bg2.reshape(1, outd))

    out = pl.pallas_call(
        _spmm2_body,
        grid=(g,),
        in_specs=[pl.BlockSpec((br, n), lambda i: (i, 0)),
                  pl.BlockSpec((n, br), lambda i: (0, i)),
                  pl.BlockSpec((n, outd), lambda i: (0, 0)),
                  pl.BlockSpec((br, 1), lambda i: (i, 0))],
        out_specs=pl.BlockSpec((br, outd), lambda i: (i, 0)),
        out_shape=jax.ShapeDtypeStruct((n, outd), jnp.float32),
    )(st, st, mu2, u)

    return out


# speedup vs baseline: 10.8210x; 2.1370x over previous
import jax
import jax.numpy as jnp
from jax.experimental import pallas as pl

_K = 21
_INL = 6.0
_NEG = -3.0e38


def _emb_body(x_ref, w1_ref, w2_ref, emb_ref):
    h = jnp.tanh(x_ref[...] * w1_ref[...]) * w2_ref[...]
    nrm = jnp.sqrt(jnp.sum(h * h, axis=1, keepdims=True))
    emb_ref[...] = h / jnp.maximum(nrm, 1e-12)


def _simtopk_body(emb_blk_ref, emb_all_ref, st_ref, rowsum_ref, colsum_ref):
    i = pl.program_id(0)
    a = emb_blk_ref[...]
    b = emb_all_ref[...]
    s0 = jax.lax.dot_general(a, b, (((1,), (1,)), ((), ())),
                             preferred_element_type=jnp.float32)
    n = s0.shape[1]
    col = jax.lax.broadcasted_iota(jnp.int32, s0.shape, 1)
    bits = jax.lax.bitcast_convert_type(s0, jnp.int32)
    imask = n - 1
    p = jax.lax.bitcast_convert_type((bits & ~imask) | (imask - col),
                                     jnp.float32)
    for _ in range(_K):
        m = jnp.max(p, axis=1, keepdims=True)
        p = jnp.where(p == m, _NEG, p)
    z = jnp.where(p == _NEG, s0, 0.0) * _INL - _INL
    st = jnp.where(z > 0, z + 1.0, jnp.exp(z))
    st_ref[...] = st
    rowsum_ref[...] = jnp.sum(st, axis=1, keepdims=True)
    onesv = jnp.full((st.shape[0], 1), 1.0, jnp.float32)
    csum = jax.lax.dot_general(st, onesv, (((0,), (0,)), ((), ())),
                               preferred_element_type=jnp.float32)

    @pl.when(i == 0)
    def _():
        colsum_ref[...] = jnp.zeros_like(colsum_ref)

    colsum_ref[...] += csum


def _prep_body(x_ref, wg1_ref, bg1_ref, rowsum_ref, colsum_ref, u_ref, mu1_ref):
    deg = 0.5 * (rowsum_ref[...] + colsum_ref[...])
    u = 1.0 / (jnp.sqrt(deg) + 1e-10)
    u_ref[...] = u
    m1 = jnp.dot(x_ref[...], wg1_ref[...],
                 preferred_element_type=jnp.float32) + bg1_ref[...]
    mu1_ref[...] = u * m1


def _spmm1_body(strow_ref, stcol_ref, mu1_ref, u_ref, wg2_ref, bg2_ref,
                mu2_ref):
    t = jax.lax.dot_general(strow_ref[...], mu1_ref[...],
                            (((1,), (0,)), ((), ())),
                            preferred_element_type=jnp.float32)
    t2 = jax.lax.dot_general(stcol_ref[...], mu1_ref[...],
                             (((0,), (0,)), ((), ())),
                             preferred_element_type=jnp.float32)
    h1 = jnp.maximum(u_ref[...] * (0.5 * (t + t2)), 0.0)
    m2 = jnp.dot(h1, wg2_ref[...],
                 preferred_element_type=jnp.float32) + bg2_ref[...]
    mu2_ref[...] = u_ref[...] * m2


def _spmm2_body(strow_ref, stcol_ref, mu2_ref, u_ref, out_ref):
    t = jax.lax.dot_general(strow_ref[...], mu2_ref[...],
                            (((1,), (0,)), ((), ())),
                            preferred_element_type=jnp.float32)
    t2 = jax.lax.dot_general(stcol_ref[...], mu2_ref[...],
                             (((0,), (0,)), ((), ())),
                             preferred_element_type=jnp.float32)
    out_ref[...] = u_ref[...] * (0.5 * (t + t2))


def kernel(x, w1, w2, Wg1, bg1, Wg2, bg2):
    n, d = x.shape
    hid = Wg1.shape[1]
    outd = Wg2.shape[1]
    br = 256 if n % 256 == 0 else n
    g = n // br

    emb = pl.pallas_call(
        _emb_body,
        out_shape=jax.ShapeDtypeStruct((n, d), jnp.float32),
    )(x, w1.reshape(1, d), w2.reshape(1, d))

    st, rowsum, colsum = pl.pallas_call(
        _simtopk_body,
        grid=(g,),
        in_specs=[pl.BlockSpec((br, d), lambda i: (i, 0)),
                  pl.BlockSpec((n, d), lambda i: (0, 0))],
        out_specs=[pl.BlockSpec((br, n), lambda i: (i, 0)),
                   pl.BlockSpec((br, 1), lambda i: (i, 0)),
                   pl.BlockSpec((n, 1), lambda i: (0, 0))],
        out_shape=[jax.ShapeDtypeStruct((n, n), jnp.float32),
                   jax.ShapeDtypeStruct((n, 1), jnp.float32),
                   jax.ShapeDtypeStruct((n, 1), jnp.float32)],
    )(emb, emb)

    u, mu1 = pl.pallas_call(
        _prep_body,
        out_shape=[jax.ShapeDtypeStruct((n, 1), jnp.float32),
                   jax.ShapeDtypeStruct((n, hid), jnp.float32)],
    )(x, Wg1, bg1.reshape(1, hid), rowsum, colsum)

    mu2 = pl.pallas_call(
        _spmm1_body,
        grid=(g,),
        in_specs=[pl.BlockSpec((br, n), lambda i: (i, 0)),
                  pl.BlockSpec((n, br), lambda i: (0, i)),
                  pl.BlockSpec((n, hid), lambda i: (0, 0)),
                  pl.BlockSpec((br, 1), lambda i: (i, 0)),
                  pl.BlockSpec((hid, outd), lambda i: (0, 0)),
                  pl.BlockSpec((1, outd), lambda i: (0, 0))],
        out_specs=pl.BlockSpec((br, outd), lambda i: (i, 0)),
        out_shape=jax.ShapeDtypeStruct((n, outd), jnp.float32),
    )(st, st, mu1, u, Wg2, bg2.reshape(1, outd))

    out = pl.pallas_call(
        _spmm2_body,
        grid=(g,),
        in_specs=[pl.BlockSpec((br, n), lambda i: (i, 0)),
                  pl.BlockSpec((n, br), lambda i: (0, i)),
                  pl.BlockSpec((n, outd), lambda i: (0, 0)),
                  pl.BlockSpec((br, 1), lambda i: (i, 0))],
        out_specs=pl.BlockSpec((br, outd), lambda i: (i, 0)),
        out_shape=jax.ShapeDtypeStruct((n, outd), jnp.float32),
    )(st, st, mu2, u)

    return out
